# baseline (speedup 1.0000x reference)
import jax
import jax.numpy as jnp
from jax.experimental import pallas as pl
from jax.experimental.pallas import tpu as pltpu

B, S, D, E = 4, 4096, 2048, 16
TOP_K = 2
CHUNK = 512
NS = S // CHUNK


def _router_kernel(x_ref, w_ref, b_ref, tw_ref, ti_ref, aw_ref, acc_ref):
    bi = pl.program_id(0)
    si = pl.program_id(1)

    part = jnp.sum(x_ref[0], axis=0)

    @pl.when(si == 0)
    def _init():
        acc_ref[bi, :] = part

    @pl.when(si != 0)
    def _acc():
        acc_ref[bi, :] = acc_ref[bi, :] + part

    @pl.when((bi == B - 1) & (si == NS - 1))
    def _finalize():
        q_pool = acc_ref[...] * (1.0 / S)
        logits = jnp.dot(q_pool, w_ref[...],
                         preferred_element_type=jnp.float32) + b_ref[0]
        m = jnp.max(logits, axis=-1, keepdims=True)
        ex = jnp.exp(logits - m)
        aw = ex / jnp.sum(ex, axis=-1, keepdims=True)
        aw_ref[...] = aw

        cols = jax.lax.broadcasted_iota(jnp.int32, (B, E), 1)
        i1 = jnp.argmax(aw, axis=-1).astype(jnp.int32)
        v1 = jnp.max(aw, axis=-1)
        masked = jnp.where(cols == i1[:, None], -jnp.inf, aw)
        i2 = jnp.argmax(masked, axis=-1).astype(jnp.int32)
        v2 = jnp.max(masked, axis=-1)
        norm = 1.0 / (v1 + v2 + 1e-10)
        tw_ref[...] = jnp.stack([v1 * norm, v2 * norm], axis=-1)
        ti_ref[...] = jnp.stack([i1, i2], axis=-1)


@jax.jit
def kernel(x_f, W, b):
    b2 = b.reshape(1, E)
    out = pl.pallas_call(
        _router_kernel,
        grid=(B, NS),
        in_specs=[
            pl.BlockSpec((1, CHUNK, D), lambda bi, si: (bi, si, 0)),
            pl.BlockSpec((D, E), lambda bi, si: (0, 0)),
            pl.BlockSpec((1, E), lambda bi, si: (0, 0)),
        ],
        out_specs=[
            pl.BlockSpec((B, TOP_K), lambda bi, si: (0, 0)),
            pl.BlockSpec((B, TOP_K), lambda bi, si: (0, 0)),
            pl.BlockSpec((B, E), lambda bi, si: (0, 0)),
        ],
        out_shape=[
            jax.ShapeDtypeStruct((B, TOP_K), jnp.float32),
            jax.ShapeDtypeStruct((B, TOP_K), jnp.int32),
            jax.ShapeDtypeStruct((B, E), jnp.float32),
        ],
        scratch_shapes=[pltpu.VMEM((B, D), jnp.float32)],
    )(x_f, W, b2)
    return tuple(out)
